# Initial kernel scaffold; baseline (speedup 1.0000x reference)
import jax
import jax.numpy as jnp
from jax.experimental import pallas as pl


def kernel(x, edge_index, layer_idx, Wlp, Wrp, bp, g_top, b_top, Wl0, Wr0, bb0, g0, be0, Wl1, Wr1, bb1, g1, be1, Wl2, Wr2, bb2, g2, be2, Wlf, Wrf, bf):
    raise NotImplementedError("write your pallas kernel here")


# speedup vs baseline: 4.1168x; 4.1168x over previous
import functools

import jax
import jax.numpy as jnp
from jax import lax
from jax.experimental import pallas as pl
from jax.experimental.pallas import tpu as pltpu
from jax.experimental.pallas import tpu_sc as plsc

_EPS = 1e-5

_NC = 2
_NS = 16
_NW = _NC * _NS
_CH = 80
_RPT = 632
_BT = 1000

_HI = lax.Precision.HIGHEST


def _tile_copy(s, src_at, dst_at, n):
    tail = n - (_NS - 1) * _RPT

    @pl.when(s < _NS - 1)
    def _():
        pltpu.sync_copy(src_at(s * _RPT, _RPT), dst_at(s * _RPT, _RPT))

    @pl.when(s == _NS - 1)
    def _():
        pltpu.sync_copy(src_at((_NS - 1) * _RPT, tail),
                        dst_at((_NS - 1) * _RPT, tail))


def _make_agg(n, e, d):
    epw = e // _NW
    nch = epw // _CH
    mesh = plsc.VectorSubcoreMesh(core_axis_name="c", subcore_axis_name="s")

    @functools.partial(
        pl.kernel,
        mesh=mesh,
        out_type=jax.ShapeDtypeStruct((_NC * n, d), jnp.float32),
        scratch_types=[
            pltpu.VMEM((_CH,), jnp.int32),
            pltpu.VMEM((_CH,), jnp.int32),
            pltpu.VMEM((_CH, d), jnp.float32),
            pltpu.VMEM_SHARED((n, d), jnp.float32),
            pltpu.SemaphoreType.DMA,
        ],
    )
    def agg(table_hbm, src_hbm, dst_hbm, zero_hbm, out_hbm,
            src_v, dst_v, rows_v, acc_sh, sem):
        c = lax.axis_index("c")
        s = lax.axis_index("s")
        _tile_copy(s, lambda o, l: zero_hbm.at[pl.ds(0, l)],
                   lambda o, l: acc_sh.at[pl.ds(o, l)], n)
        plsc.subcore_barrier()
        base = (c * _NS + s) * epw

        def body(i, carry):
            off = base + i * _CH
            pltpu.sync_copy(src_hbm.at[pl.ds(off, _CH)], src_v)
            pltpu.sync_copy(dst_hbm.at[pl.ds(off, _CH)], dst_v)
            pltpu.async_copy(table_hbm.at[src_v], rows_v, sem).wait()
            pltpu.sync_copy(rows_v, acc_sh.at[dst_v], add=True)
            return carry

        lax.fori_loop(0, nch, body, 0)
        plsc.subcore_barrier()
        _tile_copy(s, lambda o, l: acc_sh.at[pl.ds(o, l)],
                   lambda o, l: out_hbm.at[pl.ds(c * n + o, l)], n)

    return agg


def _make_count(n, e):
    epw = e // _NW
    nch = epw // _CH
    mesh = plsc.VectorSubcoreMesh(core_axis_name="c", subcore_axis_name="s")

    @functools.partial(
        pl.kernel,
        mesh=mesh,
        out_type=jax.ShapeDtypeStruct((_NC * n, 128), jnp.float32),
        scratch_types=[
            pltpu.VMEM((_CH,), jnp.int32),
            pltpu.VMEM((_CH, 128), jnp.float32),
            pltpu.VMEM_SHARED((n, 128), jnp.float32),
        ],
    )
    def cnt_k(dst_hbm, ones_hbm, zero_hbm, out_hbm, dst_v, ones_v, acc_sh):
        c = lax.axis_index("c")
        s = lax.axis_index("s")
        pltpu.sync_copy(ones_hbm, ones_v)
        _tile_copy(s, lambda o, l: zero_hbm.at[pl.ds(0, l)],
                   lambda o, l: acc_sh.at[pl.ds(o, l)], n)
        plsc.subcore_barrier()
        base = (c * _NS + s) * epw

        def body(i, carry):
            off = base + i * _CH
            pltpu.sync_copy(dst_hbm.at[pl.ds(off, _CH)], dst_v)
            pltpu.sync_copy(ones_v, acc_sh.at[dst_v], add=True)
            return carry

        lax.fori_loop(0, nch, body, 0)
        plsc.subcore_barrier()
        _tile_copy(s, lambda o, l: acc_sh.at[pl.ds(o, l)],
                   lambda o, l: out_hbm.at[pl.ds(c * n + o, l)], n)

    return cnt_k


def _tc_inv(cnt_ref, inv_ref):
    c = cnt_ref[0, :, 0:1] + cnt_ref[1, :, 0:1]
    inv_ref[...] = 1.0 / jnp.maximum(c, 1.0)


def _prep_inv(cntp, n):
    c3 = cntp.reshape(2, n, 128)
    return pl.pallas_call(
        _tc_inv,
        grid=(n // _BT,),
        in_specs=[pl.BlockSpec((2, _BT, 128), lambda i: (0, i, 0))],
        out_specs=pl.BlockSpec((_BT, 1), lambda i: (i, 0)),
        out_shape=jax.ShapeDtypeStruct((n, 1), jnp.float32),
    )(c3)


def _tc_mm(p_ref, inv_ref, h_ref, wl_ref, wr_ref, b_ref, u_ref, st_ref):
    i = pl.program_id(0)
    a = (p_ref[0] + p_ref[1]) * inv_ref[...]
    t = (jnp.dot(a, wl_ref[...], preferred_element_type=jnp.float32,
                 precision=_HI)
         + jnp.dot(h_ref[...], wr_ref[...], preferred_element_type=jnp.float32,
                   precision=_HI)
         + b_ref[...])
    t = jnp.maximum(t, 0.0)
    u_ref[...] = t
    s1 = jnp.sum(t, axis=0, keepdims=True)
    s2 = jnp.sum(t * t, axis=0, keepdims=True)

    @pl.when(i == 0)
    def _():
        st_ref[0:1, :] = s1
        st_ref[1:2, :] = s2

    @pl.when(i > 0)
    def _():
        st_ref[0:1, :] = st_ref[0:1, :] + s1
        st_ref[1:2, :] = st_ref[1:2, :] + s2


def _make_tc_bn(n):
    inv_n = 1.0 / n

    def _tc_bn(u_ref, st_ref, g_ref, be_ref, o_ref):
        m = st_ref[0:1, :] * inv_n
        v = st_ref[1:2, :] * inv_n - m * m
        o_ref[...] = ((u_ref[...] - m) / jnp.sqrt(v + _EPS) * g_ref[...]
                      + be_ref[...])

    return _tc_bn


def _tc_fin(p_ref, inv_ref, h_ref, wl_ref, wr_ref, b_ref, o_ref):
    a = (p_ref[0] + p_ref[1]) * inv_ref[...]
    t = (jnp.dot(a, wl_ref[...], preferred_element_type=jnp.float32,
                 precision=_HI)
         + jnp.dot(h_ref[...], wr_ref[...], preferred_element_type=jnp.float32,
                   precision=_HI)
         + b_ref[...])
    t = t - jnp.max(t, axis=1, keepdims=True)
    o_ref[...] = t - jnp.log(jnp.sum(jnp.exp(t), axis=1, keepdims=True))


def _dense_hidden(p, inv, h, wl, wr, b, g, be):
    n, d = h.shape
    nb = n // _BT
    p3 = p.reshape(2, n, d)
    u, st = pl.pallas_call(
        _tc_mm,
        grid=(nb,),
        in_specs=[
            pl.BlockSpec((2, _BT, d), lambda i: (0, i, 0)),
            pl.BlockSpec((_BT, 1), lambda i: (i, 0)),
            pl.BlockSpec((_BT, d), lambda i: (i, 0)),
            pl.BlockSpec((d, d), lambda i: (0, 0)),
            pl.BlockSpec((d, d), lambda i: (0, 0)),
            pl.BlockSpec((1, d), lambda i: (0, 0)),
        ],
        out_specs=[
            pl.BlockSpec((_BT, d), lambda i: (i, 0)),
            pl.BlockSpec((8, d), lambda i: (0, 0)),
        ],
        out_shape=[
            jax.ShapeDtypeStruct((n, d), jnp.float32),
            jax.ShapeDtypeStruct((8, d), jnp.float32),
        ],
    )(p3, inv, h, wl, wr, b.reshape(1, -1))
    return pl.pallas_call(
        _make_tc_bn(n),
        grid=(nb,),
        in_specs=[
            pl.BlockSpec((_BT, d), lambda i: (i, 0)),
            pl.BlockSpec((8, d), lambda i: (0, 0)),
            pl.BlockSpec((1, d), lambda i: (0, 0)),
            pl.BlockSpec((1, d), lambda i: (0, 0)),
        ],
        out_specs=pl.BlockSpec((_BT, d), lambda i: (i, 0)),
        out_shape=jax.ShapeDtypeStruct((n, d), jnp.float32),
    )(u, st, g.reshape(1, -1), be.reshape(1, -1))


def _dense_final(p, inv, h, wl, wr, b):
    n, d = h.shape
    cc = wl.shape[1]
    nb = n // _BT
    p3 = p.reshape(2, n, d)
    return pl.pallas_call(
        _tc_fin,
        grid=(nb,),
        in_specs=[
            pl.BlockSpec((2, _BT, d), lambda i: (0, i, 0)),
            pl.BlockSpec((_BT, 1), lambda i: (i, 0)),
            pl.BlockSpec((_BT, d), lambda i: (i, 0)),
            pl.BlockSpec((d, cc), lambda i: (0, 0)),
            pl.BlockSpec((d, cc), lambda i: (0, 0)),
            pl.BlockSpec((1, cc), lambda i: (0, 0)),
        ],
        out_specs=pl.BlockSpec((_BT, cc), lambda i: (i, 0)),
        out_shape=jax.ShapeDtypeStruct((n, cc), jnp.float32),
    )(p3, inv, h, wl, wr, b.reshape(1, -1))


def kernel(x, edge_index, layer_idx, Wlp, Wrp, bp, g_top, b_top,
           Wl0, Wr0, bb0, g0, be0,
           Wl1, Wr1, bb1, g1, be1,
           Wl2, Wr2, bb2, g2, be2,
           Wlf, Wrf, bf):
    n, d = x.shape
    e = edge_index.shape[1]
    src = edge_index[0].astype(jnp.int32)
    dst = edge_index[1].astype(jnp.int32)

    zero_d = jnp.zeros((_RPT, d), jnp.float32)
    ones_c = jnp.ones((_CH, 128), jnp.float32)

    agg = _make_agg(n, e, d)
    cntp = _make_count(n, e)(dst, ones_c, zero_d)
    inv = _prep_inv(cntp, n)

    p = agg(x, src, dst, zero_d)
    h = _dense_hidden(p, inv, x, Wlp, Wrp, bp, g_top, b_top)
    for i, (wl, wr, bb, g, be) in enumerate([
            (Wl0, Wr0, bb0, g0, be0),
            (Wl1, Wr1, bb1, g1, be1),
            (Wl2, Wr2, bb2, g2, be2)]):
        p = agg(h, src, dst, zero_d)
        h_new = _dense_hidden(p, inv, h, wl, wr, bb, g, be)
        h = jnp.where(jnp.asarray(i, jnp.int32) <= layer_idx, h_new, h)

    p = agg(h, src, dst, zero_d)
    return _dense_final(p, inv, h, Wlf, Wrf, bf)


# speedup vs baseline: 8.4624x; 2.0556x over previous
import functools

import jax
import jax.numpy as jnp
from jax import lax
from jax.experimental import pallas as pl
from jax.experimental.pallas import tpu as pltpu
from jax.experimental.pallas import tpu_sc as plsc

_EPS = 1e-5

_NC = 2
_NS = 16
_NW = _NC * _NS
_CH = 80
_RPT = 632
_BT = 1000

_HI = lax.Precision.HIGHEST


def _tile_copy(s, src_at, dst_at, n):
    tail = n - (_NS - 1) * _RPT

    @pl.when(s < _NS - 1)
    def _():
        pltpu.sync_copy(src_at(s * _RPT, _RPT), dst_at(s * _RPT, _RPT))

    @pl.when(s == _NS - 1)
    def _():
        pltpu.sync_copy(src_at((_NS - 1) * _RPT, tail),
                        dst_at((_NS - 1) * _RPT, tail))


def _make_agg(n, e, d):
    epw = e // _NW
    nch = epw // _CH
    mesh = plsc.VectorSubcoreMesh(core_axis_name="c", subcore_axis_name="s")

    @functools.partial(
        pl.kernel,
        mesh=mesh,
        out_type=jax.ShapeDtypeStruct((_NC * n, d), jnp.float32),
        scratch_types=[
            pltpu.VMEM((epw,), jnp.int32),
            pltpu.VMEM((nch, _CH), jnp.int32),
            pltpu.VMEM((_CH, d), jnp.float32),
            pltpu.VMEM((_CH, d), jnp.float32),
            pltpu.VMEM_SHARED((n, d), jnp.float32),
            pltpu.SemaphoreType.DMA,
            pltpu.SemaphoreType.DMA,
        ],
    )
    def agg(table_hbm, pk_hbm, out_hbm,
            si_v, di_v, rows0, rows1, acc_sh, sem0, sem1):
        c = lax.axis_index("c")
        s = lax.axis_index("s")
        w = c * _NS + s
        pltpu.sync_copy(pk_hbm.at[pl.ds(w * epw, epw)], si_v)

        def unrow(r, carry):
            for j in range(_CH // 16):
                off = r * _CH + 16 * j
                v = si_v[pl.ds(off, 16)]
                di_v[r, pl.ds(16 * j, 16)] = lax.shift_right_logical(v, 14)
                si_v[pl.ds(off, 16)] = lax.bitwise_and(v, 16383)
            return carry

        lax.fori_loop(0, nch, unrow, 0)

        z16 = jnp.zeros((16,), jnp.float32)

        def zrow(r, carry):
            for j in range(d // 16):
                rows0[r, pl.ds(16 * j, 16)] = z16
            return carry

        lax.fori_loop(0, _CH, zrow, 0)
        base_r = s * _RPT

        @pl.when(s < _NS - 1)
        def _():
            for m in range(_RPT // _CH):
                pltpu.sync_copy(rows0, acc_sh.at[pl.ds(base_r + m * _CH, _CH)])
            rem = _RPT % _CH
            if rem:
                pltpu.sync_copy(rows0.at[pl.ds(0, rem)],
                                acc_sh.at[pl.ds(base_r + _RPT - rem, rem)])

        @pl.when(s == _NS - 1)
        def _():
            tail = n - (_NS - 1) * _RPT
            for m in range(tail // _CH):
                pltpu.sync_copy(rows0, acc_sh.at[pl.ds(base_r + m * _CH, _CH)])
            rem = tail % _CH
            if rem:
                pltpu.sync_copy(rows0.at[pl.ds(0, rem)],
                                acc_sh.at[pl.ds(base_r + tail - rem, rem)])

        plsc.subcore_barrier()

        def start(k, buf, sem):
            pltpu.async_copy(table_hbm.at[si_v.at[pl.ds(k * _CH, _CH)]],
                             buf, sem)

        def wait(buf, sem):
            pltpu.make_async_copy(table_hbm.at[pl.ds(0, _CH)], buf, sem).wait()

        def scatter(k, buf):
            pltpu.sync_copy(buf, acc_sh.at[di_v.at[k]], add=True)

        start(0, rows0, sem0)

        def body(i, carry):
            k0 = 2 * i
            start(k0 + 1, rows1, sem1)
            wait(rows0, sem0)
            scatter(k0, rows0)
            start(k0 + 2, rows0, sem0)
            wait(rows1, sem1)
            scatter(k0 + 1, rows1)
            return carry

        lax.fori_loop(0, (nch - 1) // 2, body, 0)
        wait(rows0, sem0)
        scatter(nch - 1, rows0)
        plsc.subcore_barrier()
        _tile_copy(s, lambda o, l: acc_sh.at[pl.ds(o, l)],
                   lambda o, l: out_hbm.at[pl.ds(c * n + o, l)], n)

    return agg


def _make_count(n, e):
    epw = e // _NW
    nch = epw // _CH
    mesh = plsc.VectorSubcoreMesh(core_axis_name="c", subcore_axis_name="s")

    @functools.partial(
        pl.kernel,
        mesh=mesh,
        out_type=jax.ShapeDtypeStruct((_NC * n, 128), jnp.float32),
        scratch_types=[
            pltpu.VMEM((_CH,), jnp.int32),
            pltpu.VMEM((_CH, 128), jnp.float32),
            pltpu.VMEM_SHARED((n, 128), jnp.float32),
        ],
    )
    def cnt_k(dst_hbm, ones_hbm, zero_hbm, out_hbm, dst_v, ones_v, acc_sh):
        c = lax.axis_index("c")
        s = lax.axis_index("s")
        pltpu.sync_copy(ones_hbm, ones_v)
        _tile_copy(s, lambda o, l: zero_hbm.at[pl.ds(0, l)],
                   lambda o, l: acc_sh.at[pl.ds(o, l)], n)
        plsc.subcore_barrier()
        base = (c * _NS + s) * epw

        def body(i, carry):
            off = base + i * _CH
            pltpu.sync_copy(dst_hbm.at[pl.ds(off, _CH)], dst_v)
            pltpu.sync_copy(ones_v, acc_sh.at[dst_v], add=True)
            return carry

        lax.fori_loop(0, nch, body, 0)
        plsc.subcore_barrier()
        _tile_copy(s, lambda o, l: acc_sh.at[pl.ds(o, l)],
                   lambda o, l: out_hbm.at[pl.ds(c * n + o, l)], n)

    return cnt_k


def _tc_inv(cnt_ref, inv_ref):
    c = cnt_ref[0, :, 0:1] + cnt_ref[1, :, 0:1]
    inv_ref[...] = 1.0 / jnp.maximum(c, 1.0)


def _prep_inv(cntp, n):
    c3 = cntp.reshape(2, n, 128)
    return pl.pallas_call(
        _tc_inv,
        grid=(n // _BT,),
        in_specs=[pl.BlockSpec((2, _BT, 128), lambda i: (0, i, 0))],
        out_specs=pl.BlockSpec((_BT, 1), lambda i: (i, 0)),
        out_shape=jax.ShapeDtypeStruct((n, 1), jnp.float32),
    )(c3)


def _tc_mm(p_ref, inv_ref, h_ref, wl_ref, wr_ref, b_ref, u_ref, st_ref):
    i = pl.program_id(0)
    a = (p_ref[0] + p_ref[1]) * inv_ref[...]
    t = (jnp.dot(a, wl_ref[...], preferred_element_type=jnp.float32,
                 precision=_HI)
         + jnp.dot(h_ref[...], wr_ref[...], preferred_element_type=jnp.float32,
                   precision=_HI)
         + b_ref[...])
    t = jnp.maximum(t, 0.0)
    u_ref[...] = t
    s1 = jnp.sum(t, axis=0, keepdims=True)
    s2 = jnp.sum(t * t, axis=0, keepdims=True)

    @pl.when(i == 0)
    def _():
        st_ref[0:1, :] = s1
        st_ref[1:2, :] = s2

    @pl.when(i > 0)
    def _():
        st_ref[0:1, :] = st_ref[0:1, :] + s1
        st_ref[1:2, :] = st_ref[1:2, :] + s2


def _make_tc_bn(n):
    inv_n = 1.0 / n

    def _tc_bn(u_ref, st_ref, g_ref, be_ref, o_ref):
        m = st_ref[0:1, :] * inv_n
        v = st_ref[1:2, :] * inv_n - m * m
        o_ref[...] = ((u_ref[...] - m) / jnp.sqrt(v + _EPS) * g_ref[...]
                      + be_ref[...])

    return _tc_bn


def _tc_fin(p_ref, inv_ref, h_ref, wl_ref, wr_ref, b_ref, o_ref):
    a = (p_ref[0] + p_ref[1]) * inv_ref[...]
    t = (jnp.dot(a, wl_ref[...], preferred_element_type=jnp.float32,
                 precision=_HI)
         + jnp.dot(h_ref[...], wr_ref[...], preferred_element_type=jnp.float32,
                   precision=_HI)
         + b_ref[...])
    t = t - jnp.max(t, axis=1, keepdims=True)
    o_ref[...] = t - jnp.log(jnp.sum(jnp.exp(t), axis=1, keepdims=True))


def _dense_hidden(p, inv, h, wl, wr, b, g, be):
    n, d = h.shape
    nb = n // _BT
    p3 = p.reshape(2, n, d)
    u, st = pl.pallas_call(
        _tc_mm,
        grid=(nb,),
        in_specs=[
            pl.BlockSpec((2, _BT, d), lambda i: (0, i, 0)),
            pl.BlockSpec((_BT, 1), lambda i: (i, 0)),
            pl.BlockSpec((_BT, d), lambda i: (i, 0)),
            pl.BlockSpec((d, d), lambda i: (0, 0)),
            pl.BlockSpec((d, d), lambda i: (0, 0)),
            pl.BlockSpec((1, d), lambda i: (0, 0)),
        ],
        out_specs=[
            pl.BlockSpec((_BT, d), lambda i: (i, 0)),
            pl.BlockSpec((8, d), lambda i: (0, 0)),
        ],
        out_shape=[
            jax.ShapeDtypeStruct((n, d), jnp.float32),
            jax.ShapeDtypeStruct((8, d), jnp.float32),
        ],
    )(p3, inv, h, wl, wr, b.reshape(1, -1))
    return pl.pallas_call(
        _make_tc_bn(n),
        grid=(nb,),
        in_specs=[
            pl.BlockSpec((_BT, d), lambda i: (i, 0)),
            pl.BlockSpec((8, d), lambda i: (0, 0)),
            pl.BlockSpec((1, d), lambda i: (0, 0)),
            pl.BlockSpec((1, d), lambda i: (0, 0)),
        ],
        out_specs=pl.BlockSpec((_BT, d), lambda i: (i, 0)),
        out_shape=jax.ShapeDtypeStruct((n, d), jnp.float32),
    )(u, st, g.reshape(1, -1), be.reshape(1, -1))


def _dense_final(p, inv, h, wl, wr, b):
    n, d = h.shape
    cc = wl.shape[1]
    nb = n // _BT
    p3 = p.reshape(2, n, d)
    return pl.pallas_call(
        _tc_fin,
        grid=(nb,),
        in_specs=[
            pl.BlockSpec((2, _BT, d), lambda i: (0, i, 0)),
            pl.BlockSpec((_BT, 1), lambda i: (i, 0)),
            pl.BlockSpec((_BT, d), lambda i: (i, 0)),
            pl.BlockSpec((d, cc), lambda i: (0, 0)),
            pl.BlockSpec((d, cc), lambda i: (0, 0)),
            pl.BlockSpec((1, cc), lambda i: (0, 0)),
        ],
        out_specs=pl.BlockSpec((_BT, cc), lambda i: (i, 0)),
        out_shape=jax.ShapeDtypeStruct((n, cc), jnp.float32),
    )(p3, inv, h, wl, wr, b.reshape(1, -1))


def kernel(x, edge_index, layer_idx, Wlp, Wrp, bp, g_top, b_top,
           Wl0, Wr0, bb0, g0, be0,
           Wl1, Wr1, bb1, g1, be1,
           Wl2, Wr2, bb2, g2, be2,
           Wlf, Wrf, bf):
    n, d = x.shape
    e = edge_index.shape[1]
    src = edge_index[0].astype(jnp.int32)
    dst = edge_index[1].astype(jnp.int32)

    zero_d = jnp.zeros((_RPT, d), jnp.float32)
    ones_c = jnp.ones((_CH, 128), jnp.float32)

    agg = _make_agg(n, e, d)
    cntp = _make_count(n, e)(dst, ones_c, zero_d)
    inv = _prep_inv(cntp, n)

    pk3 = (dst << 14) | src

    p = agg(x, pk3)
    h = _dense_hidden(p, inv, x, Wlp, Wrp, bp, g_top, b_top)
    for i, (wl, wr, bb, g, be) in enumerate([
            (Wl0, Wr0, bb0, g0, be0),
            (Wl1, Wr1, bb1, g1, be1),
            (Wl2, Wr2, bb2, g2, be2)]):
        p = agg(h, pk3)
        h_new = _dense_hidden(p, inv, h, wl, wr, bb, g, be)
        h = jnp.where(jnp.asarray(i, jnp.int32) <= layer_idx, h_new, h)

    p = agg(h, pk3)
    return _dense_final(p, inv, h, Wlf, Wrf, bf)
